# Initial kernel scaffold; baseline (speedup 1.0000x reference)
import jax
import jax.numpy as jnp
from jax.experimental import pallas as pl


def kernel(query, psi_prime, knn_indices, mask, current_coords, all_coords, demands, lambda_param, mu_param, nu_param):
    raise NotImplementedError("write your pallas kernel here")


# speedup vs baseline: 3.2906x; 3.2906x over previous
import functools

import jax
import jax.numpy as jnp
from jax import lax
from jax.experimental import pallas as pl
from jax.experimental.pallas import tpu as pltpu
from jax.experimental.pallas import tpu_sc as plsc

B, NP1, D, K = 64, 2048, 128, 16
NC, NS = 2, 16
NW = NC * NS
NODES = B * NP1
NPW = NODES // NW
CH = 8
NCHUNKS = NPW // CH
LANES = 16


def _sc_partial_dots(psi_flat, idx_flat):
  mesh = plsc.VectorSubcoreMesh(
      core_axis_name="c", subcore_axis_name="s",
      num_cores=NC, num_subcores=NS)

  @functools.partial(
      pl.kernel,
      out_type=jax.ShapeDtypeStruct((NODES, LANES), jnp.float32),
      mesh=mesh,
      scratch_types=[
          pltpu.VMEM((CH * K,), jnp.int32),
          pltpu.VMEM((CH * K, D), jnp.float32),
          pltpu.VMEM((CH, D), jnp.float32),
          pltpu.VMEM((CH, LANES), jnp.float32),
          pltpu.SemaphoreType.DMA,
      ],
  )
  def body(psi_hbm, idx_hbm, out_hbm, idx_v, rows_v, own_v, outp_v, sem):
    wid = lax.axis_index("s") * NC + lax.axis_index("c")
    node0 = wid * NPW

    def chunk_body(ci, carry):
      nbase = node0 + ci * CH
      pltpu.sync_copy(idx_hbm.at[pl.ds(nbase * K, CH * K)], idx_v)
      pltpu.async_copy(psi_hbm.at[idx_v], rows_v, sem).wait()
      pltpu.sync_copy(psi_hbm.at[pl.ds(nbase, CH)], own_v)

      def node_body(i, carry2):
        zero = jnp.zeros((LANES,), jnp.float32)
        accs0 = (zero,) * (D // LANES)

        def k_body(k, accs):
          r = i * K + k
          return tuple(
              accs[j] + rows_v[r, pl.ds(j * LANES, LANES)]
              for j in range(D // LANES))

        accs = lax.fori_loop(0, K, k_body, accs0)
        p = zero
        for j in range(D // LANES):
          p = p + accs[j] * own_v[i, pl.ds(j * LANES, LANES)]
        outp_v[i, :] = p
        return carry2

      lax.fori_loop(0, CH, node_body, 0)
      pltpu.sync_copy(outp_v, out_hbm.at[pl.ds(nbase, CH)])
      return carry

    lax.fori_loop(0, NCHUNKS, chunk_body, 0)

  return body(psi_flat, idx_flat)


def _tc_combine_body(psi_ref, q_ref, cc_ref, ac_ref, dem_ref, mask_ref,
                     part_ref, lam_ref, mu_ref, nu_ref, out_ref):
  psi = psi_ref[0]
  q = q_ref[0, 0]
  context = jnp.sum(psi * q[None, :], axis=1)
  acx = ac_ref[0, 0, :]
  acy = ac_ref[0, 1, :]
  dx = acx - cc_ref[0, 0, 0]
  dy = acy - cc_ref[0, 0, 1]
  dist = jnp.sqrt(dx * dx + dy * dy)
  interf = jnp.sum(part_ref[0], axis=1)
  scores = (context + lam_ref[0, 0] * interf - mu_ref[0, 0] * dist
            + nu_ref[0, 0] * dem_ref[0, 0])
  scores = jnp.where(mask_ref[0, 0], -1000000000.0, scores)
  m = jnp.max(scores)
  shifted = scores - m
  lse = jnp.log(jnp.sum(jnp.exp(shifted)))
  out_ref[0, 0] = shifted - lse


def kernel(query, psi_prime, knn_indices, mask, current_coords, all_coords,
           demands, lambda_param, mu_param, nu_param):
  psi_flat = psi_prime.reshape(NODES, D)
  base = (jnp.arange(B, dtype=jnp.int32) * NP1)[:, None, None]
  idx_flat = (knn_indices + base).reshape(NODES * K)

  partial = _sc_partial_dots(psi_flat, idx_flat)
  partial = partial.reshape(B, NP1, LANES)

  lam_eff = jnp.clip(lambda_param, -2.0, 3.0).reshape(1, 1)
  mu_eff = jnp.clip(mu_param, 0.0, 20.0).reshape(1, 1)
  nu_eff = jnp.clip(nu_param, -2.0, 3.0).reshape(1, 1)
  coords_t = all_coords.transpose(0, 2, 1)

  grid_spec = pl.GridSpec(
      grid=(B,),
      in_specs=[
          pl.BlockSpec((1, NP1, D), lambda b: (b, 0, 0)),
          pl.BlockSpec((1, 1, D), lambda b: (b, 0, 0)),
          pl.BlockSpec((1, 1, 2), lambda b: (b, 0, 0),
                       memory_space=pltpu.SMEM),
          pl.BlockSpec((1, 2, NP1), lambda b: (b, 0, 0)),
          pl.BlockSpec((1, 1, NP1), lambda b: (b, 0, 0)),
          pl.BlockSpec((1, 1, NP1), lambda b: (b, 0, 0)),
          pl.BlockSpec((1, NP1, LANES), lambda b: (b, 0, 0)),
          pl.BlockSpec((1, 1), lambda b: (0, 0), memory_space=pltpu.SMEM),
          pl.BlockSpec((1, 1), lambda b: (0, 0), memory_space=pltpu.SMEM),
          pl.BlockSpec((1, 1), lambda b: (0, 0), memory_space=pltpu.SMEM),
      ],
      out_specs=pl.BlockSpec((1, 1, NP1), lambda b: (b, 0, 0)),
  )
  log_probs = pl.pallas_call(
      _tc_combine_body,
      grid_spec=grid_spec,
      out_shape=jax.ShapeDtypeStruct((B, 1, NP1), jnp.float32),
  )(psi_prime, query.reshape(B, 1, D), current_coords.reshape(B, 1, 2), coords_t,
    demands.reshape(B, 1, NP1), mask.reshape(B, 1, NP1), partial,
    lam_eff, mu_eff, nu_eff)
  return log_probs.reshape(B, NP1)


# speedup vs baseline: 7.3251x; 2.2261x over previous
import functools

import jax
import jax.numpy as jnp
from jax import lax
from jax.experimental import pallas as pl
from jax.experimental.pallas import tpu as pltpu
from jax.experimental.pallas import tpu_sc as plsc

B, NP1, D, K = 64, 2048, 128, 16
NC, NS = 2, 16
NW = NC * NS
NODES = B * NP1
NPW = NODES // NW
CH = 8
NCHUNKS = NPW // CH
LANES = 16


SLAB = 32
NJ = D // LANES


def _sc_partial_dots(psi_flat, idx_flat):
  mesh = plsc.VectorSubcoreMesh(
      core_axis_name="c", subcore_axis_name="s",
      num_cores=NC, num_subcores=NS)

  nslabs = NCHUNKS // SLAB
  slab_idx = SLAB * CH * K
  slab_nodes = SLAB * CH

  @functools.partial(
      pl.kernel,
      out_type=jax.ShapeDtypeStruct((NODES, LANES), jnp.float32),
      mesh=mesh,
      scratch_types=[
          pltpu.VMEM((slab_idx,), jnp.int32),
          pltpu.VMEM((slab_idx,), jnp.int32),
          pltpu.VMEM((CH * K, D), jnp.float32),
          pltpu.VMEM((CH * K, D), jnp.float32),
          pltpu.VMEM((CH, D), jnp.float32),
          pltpu.VMEM((CH, D), jnp.float32),
          pltpu.VMEM((slab_nodes, LANES), jnp.float32),
          pltpu.SemaphoreType.DMA,
          pltpu.SemaphoreType.DMA,
          pltpu.SemaphoreType.DMA,
          pltpu.SemaphoreType.DMA,
      ],
  )
  def body(psi_hbm, idx_hbm, out_hbm, idx0, idx1, rows_a, rows_b, own_a,
           own_b, outp_v, sem_a, sem_b, sem_i0, sem_i1):
    wid = lax.axis_index("s") * NC + lax.axis_index("c")
    node0 = wid * NPW

    def issue(ci, idx_v, rows_v, own_v, sem):
      off = (ci % SLAB) * CH * K
      pltpu.async_copy(
          psi_hbm.at[idx_v.at[pl.ds(off, CH * K)]], rows_v, sem)
      pltpu.async_copy(psi_hbm.at[pl.ds(node0 + ci * CH, CH)], own_v, sem)

    def wait(rows_v, own_v, sem):
      pltpu.make_async_copy(psi_hbm.at[pl.ds(0, CH * K)], rows_v, sem).wait()
      pltpu.make_async_copy(psi_hbm.at[pl.ds(0, CH)], own_v, sem).wait()

    def compute(ci, rows_v, own_v):
      srow = (ci % SLAB) * CH

      def node_body(i, carry):
        r0 = i * K
        accs = [rows_v[r0, pl.ds(j * LANES, LANES)] for j in range(NJ)]
        for k in range(1, K):
          for j in range(NJ):
            accs[j] = accs[j] + rows_v[r0 + k, pl.ds(j * LANES, LANES)]
        p = accs[0] * own_v[i, pl.ds(0, LANES)]
        for j in range(1, NJ):
          p = p + accs[j] * own_v[i, pl.ds(j * LANES, LANES)]
        outp_v[srow + i, :] = p
        return carry

      lax.fori_loop(0, CH, node_body, 0)

    def idx_fetch(h, idx_v, sem_i):
      pltpu.async_copy(
          idx_hbm.at[pl.ds(node0 * K + h * slab_idx, slab_idx)], idx_v, sem_i)

    def idx_wait(idx_v, sem_i):
      pltpu.make_async_copy(
          idx_hbm.at[pl.ds(0, slab_idx)], idx_v, sem_i).wait()

    def do_slab(h, idx_v, sem_i, idx_nv, sem_ni):
      @pl.when(h + 1 < nslabs)
      def _():
        idx_fetch(h + 1, idx_nv, sem_ni)
      idx_wait(idx_v, sem_i)
      c0 = h * SLAB
      issue(c0, idx_v, rows_a, own_a, sem_a)

      def pair_body(g, carry):
        ci = c0 + 2 * g
        issue(ci + 1, idx_v, rows_b, own_b, sem_b)
        wait(rows_a, own_a, sem_a)
        compute(ci, rows_a, own_a)

        @pl.when(g + 1 < SLAB // 2)
        def _():
          issue(ci + 2, idx_v, rows_a, own_a, sem_a)

        wait(rows_b, own_b, sem_b)
        compute(ci + 1, rows_b, own_b)
        return carry

      lax.fori_loop(0, SLAB // 2, pair_body, 0)
      pltpu.sync_copy(
          outp_v, out_hbm.at[pl.ds(node0 + h * slab_nodes, slab_nodes)])

    idx_fetch(0, idx0, sem_i0)

    def slabpair_body(s, carry):
      do_slab(2 * s, idx0, sem_i0, idx1, sem_i1)
      do_slab(2 * s + 1, idx1, sem_i1, idx0, sem_i0)
      return carry

    lax.fori_loop(0, nslabs // 2, slabpair_body, 0)

  return body(psi_flat, idx_flat)


def _tc_combine_body(psi_ref, q_ref, cc_ref, ac_ref, dem_ref, mask_ref,
                     part_ref, lam_ref, mu_ref, nu_ref, out_ref):
  psi = psi_ref[0]
  q = q_ref[0, 0]
  context = jnp.sum(psi * q[None, :], axis=1)
  acx = ac_ref[0, 0, :]
  acy = ac_ref[0, 1, :]
  dx = acx - cc_ref[0, 0, 0]
  dy = acy - cc_ref[0, 0, 1]
  dist = jnp.sqrt(dx * dx + dy * dy)
  interf = jnp.sum(part_ref[0], axis=1)
  scores = (context + lam_ref[0, 0] * interf - mu_ref[0, 0] * dist
            + nu_ref[0, 0] * dem_ref[0, 0])
  scores = jnp.where(mask_ref[0, 0], -1000000000.0, scores)
  m = jnp.max(scores)
  shifted = scores - m
  lse = jnp.log(jnp.sum(jnp.exp(shifted)))
  out_ref[0, 0] = shifted - lse


def kernel(query, psi_prime, knn_indices, mask, current_coords, all_coords,
           demands, lambda_param, mu_param, nu_param):
  psi_flat = psi_prime.reshape(NODES, D)
  base = (jnp.arange(B, dtype=jnp.int32) * NP1)[:, None, None]
  idx_flat = (knn_indices + base).reshape(NODES * K)

  partial = _sc_partial_dots(psi_flat, idx_flat)
  partial = partial.reshape(B, NP1, LANES)

  lam_eff = jnp.clip(lambda_param, -2.0, 3.0).reshape(1, 1)
  mu_eff = jnp.clip(mu_param, 0.0, 20.0).reshape(1, 1)
  nu_eff = jnp.clip(nu_param, -2.0, 3.0).reshape(1, 1)
  coords_t = all_coords.transpose(0, 2, 1)

  grid_spec = pl.GridSpec(
      grid=(B,),
      in_specs=[
          pl.BlockSpec((1, NP1, D), lambda b: (b, 0, 0)),
          pl.BlockSpec((1, 1, D), lambda b: (b, 0, 0)),
          pl.BlockSpec((1, 1, 2), lambda b: (b, 0, 0),
                       memory_space=pltpu.SMEM),
          pl.BlockSpec((1, 2, NP1), lambda b: (b, 0, 0)),
          pl.BlockSpec((1, 1, NP1), lambda b: (b, 0, 0)),
          pl.BlockSpec((1, 1, NP1), lambda b: (b, 0, 0)),
          pl.BlockSpec((1, NP1, LANES), lambda b: (b, 0, 0)),
          pl.BlockSpec((1, 1), lambda b: (0, 0), memory_space=pltpu.SMEM),
          pl.BlockSpec((1, 1), lambda b: (0, 0), memory_space=pltpu.SMEM),
          pl.BlockSpec((1, 1), lambda b: (0, 0), memory_space=pltpu.SMEM),
      ],
      out_specs=pl.BlockSpec((1, 1, NP1), lambda b: (b, 0, 0)),
  )
  log_probs = pl.pallas_call(
      _tc_combine_body,
      grid_spec=grid_spec,
      out_shape=jax.ShapeDtypeStruct((B, 1, NP1), jnp.float32),
  )(psi_prime, query.reshape(B, 1, D), current_coords.reshape(B, 1, 2), coords_t,
    demands.reshape(B, 1, NP1), mask.reshape(B, 1, NP1), partial,
    lam_eff, mu_eff, nu_eff)
  return log_probs.reshape(B, NP1)


# speedup vs baseline: 9.6454x; 1.3168x over previous
import functools

import jax
import jax.numpy as jnp
from jax import lax
from jax.experimental import pallas as pl
from jax.experimental.pallas import tpu as pltpu
from jax.experimental.pallas import tpu_sc as plsc

B, NP1, D, K = 64, 2048, 128, 16
NC, NS = 2, 16
NW = NC * NS
NODES = B * NP1
NPW = NODES // NW
CH = 8
NCHUNKS = NPW // CH
LANES = 16


SLAB = 32
NJ = D // LANES


def _sc_partial_dots(psi_flat, idx_flat):
  mesh = plsc.VectorSubcoreMesh(
      core_axis_name="c", subcore_axis_name="s",
      num_cores=NC, num_subcores=NS)

  nslabs = NCHUNKS // SLAB
  slab_idx = SLAB * CH * K
  slab_nodes = SLAB * CH

  @functools.partial(
      pl.kernel,
      out_type=jax.ShapeDtypeStruct((NODES, LANES), jnp.float32),
      mesh=mesh,
      scratch_types=[
          pltpu.VMEM((slab_idx,), jnp.int32),
          pltpu.VMEM((slab_idx,), jnp.int32),
          pltpu.VMEM((CH * K, D), jnp.float32),
          pltpu.VMEM((CH * K, D), jnp.float32),
          pltpu.VMEM((CH * K, D), jnp.float32),
          pltpu.VMEM((CH * K, D), jnp.float32),
          pltpu.VMEM((CH, D), jnp.float32),
          pltpu.VMEM((CH, D), jnp.float32),
          pltpu.VMEM((CH, D), jnp.float32),
          pltpu.VMEM((CH, D), jnp.float32),
          pltpu.VMEM((slab_nodes, LANES), jnp.float32),
          pltpu.SemaphoreType.DMA,
          pltpu.SemaphoreType.DMA,
          pltpu.SemaphoreType.DMA,
          pltpu.SemaphoreType.DMA,
          pltpu.SemaphoreType.DMA,
          pltpu.SemaphoreType.DMA,
      ],
  )
  def body(psi_hbm, idx_hbm, out_hbm, idx0, idx1, rows_0, rows_1,
           rows_2, rows_3, own_0, own_1, own_2, own_3, outp_v,
           sem_0, sem_1, sem_2, sem_3, sem_i0, sem_i1):
    rows = [rows_0, rows_1, rows_2, rows_3]
    own = [own_0, own_1, own_2, own_3]
    sems = [sem_0, sem_1, sem_2, sem_3]
    wid = lax.axis_index("s") * NC + lax.axis_index("c")
    node0 = wid * NPW

    def issue(ci, base, idx_v, rows_v, own_v, sem):
      off = (ci % SLAB) * CH * K
      for v in range(CH * K // LANES):
        sl = pl.ds(off + v * LANES, LANES)
        idx_v[sl] = idx_v[sl] + base
      pltpu.async_copy(
          psi_hbm.at[idx_v.at[pl.ds(off, CH * K)]], rows_v, sem)
      pltpu.async_copy(psi_hbm.at[pl.ds(node0 + ci * CH, CH)], own_v, sem)

    def wait(rows_v, own_v, sem):
      pltpu.make_async_copy(
          psi_hbm.at[pl.ds(0, CH * K)], rows_v, sem).wait()
      pltpu.make_async_copy(psi_hbm.at[pl.ds(0, CH)], own_v, sem).wait()

    def compute(ci, rows_v, own_v):
      srow = (ci % SLAB) * CH

      def node_body(i, carry):
        r0 = i * K
        accs = [rows_v[r0, pl.ds(j * LANES, LANES)] for j in range(NJ)]
        for k in range(1, K):
          for j in range(NJ):
            accs[j] = accs[j] + rows_v[r0 + k, pl.ds(j * LANES, LANES)]
        p = accs[0] * own_v[i, pl.ds(0, LANES)]
        for j in range(1, NJ):
          p = p + accs[j] * own_v[i, pl.ds(j * LANES, LANES)]
        outp_v[srow + i, :] = p
        return carry

      lax.fori_loop(0, CH, node_body, 0)

    def idx_fetch(h, idx_v, sem_i):
      pltpu.async_copy(
          idx_hbm.at[pl.ds(node0 * K + h * slab_idx, slab_idx)], idx_v, sem_i)

    def idx_wait(idx_v, sem_i):
      pltpu.make_async_copy(
          idx_hbm.at[pl.ds(0, slab_idx)], idx_v, sem_i).wait()

    def do_slab(h, idx_v, sem_i, idx_nv, sem_ni):
      @pl.when(h + 1 < nslabs)
      def _():
        idx_fetch(h + 1, idx_nv, sem_ni)
      idx_wait(idx_v, sem_i)
      c0 = h * SLAB
      base = (node0 + h * slab_nodes) // NP1 * NP1
      for ph in range(3):
        issue(c0 + ph, base, idx_v, rows[ph], own[ph], sems[ph])

      def quad_body(g, carry):
        for ph in range(4):
          ci = c0 + 4 * g + ph
          wait(rows[ph], own[ph], sems[ph])
          compute(ci, rows[ph], own[ph])

          @pl.when(ci + 3 < c0 + SLAB)
          def _():
            issue(ci + 3, base, idx_v, rows[(ph + 3) % 4],
                  own[(ph + 3) % 4], sems[(ph + 3) % 4])
        return carry

      lax.fori_loop(0, SLAB // 4, quad_body, 0)
      pltpu.sync_copy(
          outp_v, out_hbm.at[pl.ds(node0 + h * slab_nodes, slab_nodes)])

    idx_fetch(0, idx0, sem_i0)

    def slabpair_body(s, carry):
      do_slab(2 * s, idx0, sem_i0, idx1, sem_i1)
      do_slab(2 * s + 1, idx1, sem_i1, idx0, sem_i0)
      return carry

    lax.fori_loop(0, nslabs // 2, slabpair_body, 0)

  return body(psi_flat, idx_flat)


BB = 8


def _tc_base_body(psi_ref, q_ref, dx_ref, dy_ref, dem_ref, mask_ref,
                  mu_ref, nu_ref, out_ref):
  psi = psi_ref[...]
  q = q_ref[...]
  context = jnp.sum(psi * q, axis=2)
  dx = dx_ref[:, 0, :]
  dy = dy_ref[:, 0, :]
  dist = jnp.sqrt(dx * dx + dy * dy)
  scores = (context - mu_ref[0, 0] * dist
            + nu_ref[0, 0] * dem_ref[:, 0, :])
  out_ref[:, 0, :] = jnp.where(mask_ref[:, 0, :], -1000000000.0, scores)


def _tc_final_body(base_ref, part_ref, lam_ref, out_ref):
  interf = jnp.sum(part_ref[...], axis=2)
  scores = base_ref[:, 0, :] + lam_ref[0, 0] * interf
  m = jnp.max(scores, axis=1, keepdims=True)
  shifted = scores - m
  lse = jnp.log(jnp.sum(jnp.exp(shifted), axis=1, keepdims=True))
  out_ref[:, 0, :] = shifted - lse


def kernel(query, psi_prime, knn_indices, mask, current_coords, all_coords,
           demands, lambda_param, mu_param, nu_param):
  psi_flat = psi_prime.reshape(NODES, D)
  base = (jnp.arange(B, dtype=jnp.int32) * NP1)[:, None, None]
  idx_flat = (knn_indices + base).reshape(NODES * K)
  partial = _sc_partial_dots(psi_flat, idx_flat)
  partial = partial.reshape(B, NP1, LANES)

  lam_eff = jnp.clip(lambda_param, -2.0, 3.0).reshape(1, 1)
  mu_eff = jnp.clip(mu_param, 0.0, 20.0).reshape(1, 1)
  nu_eff = jnp.clip(nu_param, -2.0, 3.0).reshape(1, 1)
  dxy = all_coords - current_coords[:, None, :]
  dx_in = dxy[:, :, 0].reshape(B, 1, NP1)
  dy_in = dxy[:, :, 1].reshape(B, 1, NP1)

  base_spec = pl.GridSpec(
      grid=(B // BB,),
      in_specs=[
          pl.BlockSpec((BB, NP1, D), lambda b: (b, 0, 0)),
          pl.BlockSpec((BB, 1, D), lambda b: (b, 0, 0)),
          pl.BlockSpec((BB, 1, NP1), lambda b: (b, 0, 0)),
          pl.BlockSpec((BB, 1, NP1), lambda b: (b, 0, 0)),
          pl.BlockSpec((BB, 1, NP1), lambda b: (b, 0, 0)),
          pl.BlockSpec((BB, 1, NP1), lambda b: (b, 0, 0)),
          pl.BlockSpec((1, 1), lambda b: (0, 0), memory_space=pltpu.SMEM),
          pl.BlockSpec((1, 1), lambda b: (0, 0), memory_space=pltpu.SMEM),
      ],
      out_specs=pl.BlockSpec((BB, 1, NP1), lambda b: (b, 0, 0)),
  )
  base_scores = pl.pallas_call(
      _tc_base_body,
      grid_spec=base_spec,
      out_shape=jax.ShapeDtypeStruct((B, 1, NP1), jnp.float32),
  )(psi_prime, query.reshape(B, 1, D), dx_in, dy_in,
    demands.reshape(B, 1, NP1), mask.reshape(B, 1, NP1),
    mu_eff, nu_eff)

  final_spec = pl.GridSpec(
      grid=(B // BB,),
      in_specs=[
          pl.BlockSpec((BB, 1, NP1), lambda b: (b, 0, 0)),
          pl.BlockSpec((BB, NP1, LANES), lambda b: (b, 0, 0)),
          pl.BlockSpec((1, 1), lambda b: (0, 0), memory_space=pltpu.SMEM),
      ],
      out_specs=pl.BlockSpec((BB, 1, NP1), lambda b: (b, 0, 0)),
  )
  log_probs = pl.pallas_call(
      _tc_final_body,
      grid_spec=final_spec,
      out_shape=jax.ShapeDtypeStruct((B, 1, NP1), jnp.float32),
  )(base_scores, partial, lam_eff)
  return log_probs.reshape(B, NP1)
